# baseline (speedup 1.0000x reference)
import jax
import jax.numpy as jnp
from jax import lax
from jax.experimental import pallas as pl
from jax.experimental.pallas import tpu as pltpu

NCLS = 1000
B = 64
NBUF = 4


def _one_hot_body(x_ref, o_hbm, scratch, sems):
    step = pl.program_id(0)
    nsteps = pl.num_programs(0)
    slot = lax.rem(step, NBUF)

    @pl.when(step >= NBUF)
    def _():
        prev = step - NBUF
        pltpu.make_async_copy(
            scratch.at[slot],
            o_hbm.at[pl.ds(prev * B, B)],
            sems.at[slot],
        ).wait()

    x = x_ref[pl.ds(step * B, B), :]
    k = jax.lax.broadcasted_iota(jnp.int32, (B, 26, NCLS), 2)
    scratch[slot] = (k == x[:, :, None]).astype(jnp.int32)

    pltpu.make_async_copy(
        scratch.at[slot],
        o_hbm.at[pl.ds(step * B, B)],
        sems.at[slot],
    ).start()

    @pl.when(step == nsteps - 1)
    def _():
        for j in range(NBUF):
            s = step - (NBUF - 1) + j
            pltpu.make_async_copy(
                scratch.at[lax.rem(s, NBUF)],
                o_hbm.at[pl.ds(s * B, B)],
                sems.at[lax.rem(s, NBUF)],
            ).wait()


def kernel(x):
    n0, n1 = x.shape
    out = pl.pallas_call(
        _one_hot_body,
        grid=(n0 // B,),
        in_specs=[pl.BlockSpec((n0, n1), lambda i: (0, 0))],
        out_specs=pl.BlockSpec(memory_space=pl.ANY),
        out_shape=jax.ShapeDtypeStruct((n0, n1, NCLS), jnp.int32),
        scratch_shapes=[
            pltpu.VMEM((NBUF, B, n1, NCLS), jnp.int32),
            pltpu.SemaphoreType.DMA((NBUF,)),
        ],
        compiler_params=pltpu.CompilerParams(
            dimension_semantics=("arbitrary",),
        ),
    )(x)
    return out
